# baseline (speedup 1.0000x reference)
import functools

import jax
import jax.numpy as jnp
from jax import lax
from jax.experimental import pallas as pl
from jax.experimental.pallas import tpu as pltpu, tpu_sc as plsc

BATCH = 16384
DIM = 64

_info = plsc.get_sparse_core_info()
_NC, _NS = _info.num_cores, _info.num_subcores
_NW = _NC * _NS
_B_PER_W = BATCH // _NW

_mesh = plsc.VectorSubcoreMesh(core_axis_name="c", subcore_axis_name="s")


@functools.partial(
    pl.kernel,
    mesh=_mesh,
    out_type=jax.ShapeDtypeStruct((BATCH, DIM), jnp.float32),
    scratch_types=[
        pltpu.VMEM((_B_PER_W,), jnp.int32),
        pltpu.VMEM((_B_PER_W, DIM), jnp.float32),
        pltpu.SemaphoreType.DMA,
    ],
    compiler_params=pltpu.CompilerParams(use_tc_tiling_on_sc=False),
)
def _gather_kernel(idx_hbm, table_hbm, out_hbm, idx_v, rows_v, sem):
    wid = lax.axis_index("s") * _NC + lax.axis_index("c")
    base = wid * _B_PER_W
    pltpu.sync_copy(idx_hbm.at[pl.ds(base, _B_PER_W)], idx_v)
    pltpu.async_copy(table_hbm.at[idx_v], rows_v, sem).wait()
    pltpu.sync_copy(rows_v, out_hbm.at[pl.ds(base, _B_PER_W)])


@jax.jit
def kernel(indexes, entity_table):
    return _gather_kernel(indexes.astype(jnp.int32), entity_table)


# speedup vs baseline: 1.7285x; 1.7285x over previous
import functools

import jax
import jax.numpy as jnp
from jax import lax
from jax.experimental import pallas as pl
from jax.experimental.pallas import tpu as pltpu, tpu_sc as plsc

BATCH = 16384
DIM = 64

_info = plsc.get_sparse_core_info()
_NC, _NS = _info.num_cores, _info.num_subcores
_NW = _NC * _NS
_B_PER_W = BATCH // _NW

_CHUNK = 16
_N_CHUNKS = _B_PER_W // _CHUNK

_mesh = plsc.VectorSubcoreMesh(core_axis_name="c", subcore_axis_name="s")


@functools.partial(
    pl.kernel,
    mesh=_mesh,
    out_type=jax.ShapeDtypeStruct((BATCH, DIM), jnp.float32),
    scratch_types=[
        pltpu.VMEM((_B_PER_W,), jnp.int32),
        pltpu.VMEM((_B_PER_W, DIM), jnp.float32),
        pltpu.SemaphoreType.DMA,
    ],
    compiler_params=pltpu.CompilerParams(use_tc_tiling_on_sc=True),
)
def _gather_kernel(idx_hbm, table_hbm, out_hbm, idx_v, rows_v, sem):
    wid = lax.axis_index("s") * _NC + lax.axis_index("c")
    base = wid * _B_PER_W
    pltpu.sync_copy(idx_hbm.at[pl.ds(base, _B_PER_W)], idx_v)

    @pl.loop(0, _N_CHUNKS)
    def _issue(c):
        vec = idx_v[pl.ds(c * _CHUNK, _CHUNK)]
        for j in range(_CHUNK):
            r = vec[j]
            pltpu.async_copy(
                table_hbm.at[pl.ds(r, 1)],
                rows_v.at[pl.ds(c * _CHUNK + j, 1)],
                sem,
            )

    @pl.loop(0, _N_CHUNKS)
    def _drain(c):
        for j in range(_CHUNK):
            pltpu.make_async_copy(
                table_hbm.at[pl.ds(0, 1)],
                rows_v.at[pl.ds(c * _CHUNK + j, 1)],
                sem,
            ).wait()

    pltpu.sync_copy(rows_v, out_hbm.at[pl.ds(base, _B_PER_W)])


@jax.jit
def kernel(indexes, entity_table):
    return _gather_kernel(indexes.astype(jnp.int32), entity_table)
